# Initial kernel scaffold; baseline (speedup 1.0000x reference)
import jax
import jax.numpy as jnp
from jax.experimental import pallas as pl


def kernel(x, edge_index, Wl1, Wr1, att1, b1, Wl2, Wr2, att2, b2, Wl3, Wr3, att3, b3, lW1, lb1, lW2, lb2, lW3, lb3):
    raise NotImplementedError("write your pallas kernel here")


# speedup vs baseline: 1.0636x; 1.0636x over previous
import functools

import jax
import jax.numpy as jnp
from jax.experimental import pallas as pl
from jax.experimental.pallas import tpu as pltpu

N, E, D, H = 10000, 320000, 128, 512
NP = 10240
BR = 1024


def _proj_body(x_ref, w_ref, o_ref):
    o_ref[...] = jnp.dot(x_ref[...], w_ref[...], preferred_element_type=jnp.float32)


def _proj(x, w):
    K, M = w.shape
    return pl.pallas_call(
        _proj_body,
        grid=(NP // BR,),
        in_specs=[
            pl.BlockSpec((BR, K), lambda i: (i, 0)),
            pl.BlockSpec((K, M), lambda i: (0, 0)),
        ],
        out_specs=pl.BlockSpec((BR, M), lambda i: (i, 0)),
        out_shape=jax.ShapeDtypeStruct((NP, M), jnp.float32),
    )(x, w)


def _act_proj_body(g_ref, b_ref, w_ref, o_ref):
    h = jnp.tanh(g_ref[...] + b_ref[...])
    o_ref[...] = jnp.dot(h, w_ref[...], preferred_element_type=jnp.float32)


def _act_proj(g, b, w):
    K, M = w.shape
    return pl.pallas_call(
        _act_proj_body,
        grid=(NP // BR,),
        in_specs=[
            pl.BlockSpec((BR, K), lambda i: (i, 0)),
            pl.BlockSpec((1, K), lambda i: (0, 0)),
            pl.BlockSpec((K, M), lambda i: (0, 0)),
        ],
        out_specs=pl.BlockSpec((BR, M), lambda i: (i, 0)),
        out_shape=jax.ShapeDtypeStruct((NP, M), jnp.float32),
    )(g, b, w)


def _head_body(g_ref, b3_ref, w1_ref, b1_ref, w2_ref, b2_ref, w3_ref, b3p_ref, o_ref):
    h = jnp.tanh(g_ref[...] + b3_ref[...])
    h = h @ w1_ref[...] + b1_ref[...]
    h = jnp.where(h > 0, h, 0.01 * h)
    h = h @ w2_ref[...] + b2_ref[...]
    h = jnp.where(h > 0, h, 0.01 * h)
    o_ref[...] = h @ w3_ref[...] + b3p_ref[...]


def _head(g, b3, w1, b1, w2, b2, w3p, b3p):
    return pl.pallas_call(
        _head_body,
        grid=(NP // BR,),
        in_specs=[
            pl.BlockSpec((BR, H), lambda i: (i, 0)),
            pl.BlockSpec((1, H), lambda i: (0, 0)),
            pl.BlockSpec((H, H), lambda i: (0, 0)),
            pl.BlockSpec((1, H), lambda i: (0, 0)),
            pl.BlockSpec((H, H), lambda i: (0, 0)),
            pl.BlockSpec((1, H), lambda i: (0, 0)),
            pl.BlockSpec((H, 128), lambda i: (0, 0)),
            pl.BlockSpec((1, 128), lambda i: (0, 0)),
        ],
        out_specs=pl.BlockSpec((BR, 128), lambda i: (i, 0)),
        out_shape=jax.ShapeDtypeStruct((NP, 128), jnp.float32),
    )(g, b3, w1, b1, w2, b2, w3p, b3p)


def _edge_stage(xlr, att, src, dst):
    xl = xlr[:, :H]
    xr = xlr[:, H:]
    m = jax.nn.leaky_relu(xl[src] + xr[dst], 0.2)
    logits = m @ att
    seg_max = jax.ops.segment_max(logits, dst, num_segments=N)
    exp = jnp.exp(logits - seg_max[dst])
    denom = jax.ops.segment_sum(exp, dst, num_segments=N)
    alpha = exp / (denom[dst] + 1e-16)
    out = jax.ops.segment_sum(alpha[:, None] * xl[src], dst, num_segments=N)
    return jnp.pad(out, ((0, NP - N), (0, 0)))


def kernel(x, edge_index, Wl1, Wr1, att1, b1, Wl2, Wr2, att2, b2,
           Wl3, Wr3, att3, b3, lW1, lb1, lW2, lb2, lW3, lb3):
    loops = jnp.arange(N, dtype=edge_index.dtype)
    src = jnp.concatenate([edge_index[0], loops])
    dst = jnp.concatenate([edge_index[1], loops])

    xp = jnp.pad(x, ((0, NP - N), (0, 0)))
    w1 = jnp.concatenate([Wl1, Wr1], axis=1)
    w2 = jnp.concatenate([Wl2, Wr2], axis=1)
    w3 = jnp.concatenate([Wl3, Wr3], axis=1)
    w3p = jnp.pad(lW3, ((0, 0), (0, 127)))
    b3p = jnp.pad(lb3, (0, 127))

    xlr = _proj(xp, w1)
    g1 = _edge_stage(xlr, att1, src, dst)
    xlr = _act_proj(g1, b1[None, :], w2)
    g2 = _edge_stage(xlr, att2, src, dst)
    xlr = _act_proj(g2, b2[None, :], w3)
    g3 = _edge_stage(xlr, att3, src, dst)
    y = _head(g3, b3[None, :], lW1, lb1[None, :], lW2, lb2[None, :],
              w3p, b3p[None, :])
    return y[:N, :1]


# speedup vs baseline: 1.7884x; 1.6814x over previous
import functools

import jax
import jax.numpy as jnp
from jax import lax
from jax.experimental import pallas as pl
from jax.experimental.pallas import tpu as pltpu
from jax.experimental.pallas import tpu_sc as plsc

N, E, D, H = 10000, 320000, 128, 512
NP = 10240
BR = 1024
NSUB = 64
NSEG = 160
CH = 64
EPAD = 334336
HC = H // 16


def _proj_body(x_ref, w_ref, ol_ref, or_ref):
    r = jnp.dot(x_ref[...], w_ref[...], preferred_element_type=jnp.float32)
    ol_ref[...] = r[:, :H]
    or_ref[...] = r[:, H:]


def _proj(x, w):
    K = w.shape[0]
    return pl.pallas_call(
        _proj_body,
        grid=(NP // BR,),
        in_specs=[
            pl.BlockSpec((BR, K), lambda i: (i, 0)),
            pl.BlockSpec((K, 2 * H), lambda i: (0, 0)),
        ],
        out_specs=[pl.BlockSpec((BR, H), lambda i: (i, 0)),
                   pl.BlockSpec((BR, H), lambda i: (i, 0))],
        out_shape=[jax.ShapeDtypeStruct((NP, H), jnp.float32),
                   jax.ShapeDtypeStruct((NP, H), jnp.float32)],
    )(x, w)


def _act_proj_body(g_ref, b_ref, w_ref, ol_ref, or_ref):
    h = jnp.tanh(g_ref[...] + b_ref[...])
    r = jnp.dot(h, w_ref[...], preferred_element_type=jnp.float32)
    ol_ref[...] = r[:, :H]
    or_ref[...] = r[:, H:]


def _act_proj(g, b, w):
    K = w.shape[0]
    return pl.pallas_call(
        _act_proj_body,
        grid=(NP // BR,),
        in_specs=[
            pl.BlockSpec((BR, K), lambda i: (i, 0)),
            pl.BlockSpec((1, K), lambda i: (0, 0)),
            pl.BlockSpec((K, 2 * H), lambda i: (0, 0)),
        ],
        out_specs=[pl.BlockSpec((BR, H), lambda i: (i, 0)),
                   pl.BlockSpec((BR, H), lambda i: (i, 0))],
        out_shape=[jax.ShapeDtypeStruct((NP, H), jnp.float32),
                   jax.ShapeDtypeStruct((NP, H), jnp.float32)],
    )(g, b, w)


def _head_body(g_ref, b3_ref, w1_ref, b1_ref, w2_ref, b2_ref, w3_ref, b3p_ref, o_ref):
    h = jnp.tanh(g_ref[...] + b3_ref[...])
    h = h @ w1_ref[...] + b1_ref[...]
    h = jnp.where(h > 0, h, 0.01 * h)
    h = h @ w2_ref[...] + b2_ref[...]
    h = jnp.where(h > 0, h, 0.01 * h)
    o_ref[...] = h @ w3_ref[...] + b3p_ref[...]


def _head(g, b3, w1, b1, w2, b2, w3p, b3p):
    return pl.pallas_call(
        _head_body,
        grid=(NP // BR,),
        in_specs=[
            pl.BlockSpec((BR, H), lambda i: (i, 0)),
            pl.BlockSpec((1, H), lambda i: (0, 0)),
            pl.BlockSpec((H, H), lambda i: (0, 0)),
            pl.BlockSpec((1, H), lambda i: (0, 0)),
            pl.BlockSpec((H, H), lambda i: (0, 0)),
            pl.BlockSpec((1, H), lambda i: (0, 0)),
            pl.BlockSpec((H, 128), lambda i: (0, 0)),
            pl.BlockSpec((1, 128), lambda i: (0, 0)),
        ],
        out_specs=pl.BlockSpec((BR, 128), lambda i: (i, 0)),
        out_shape=jax.ShapeDtypeStruct((NP, 128), jnp.float32),
    )(g, b3, w1, b1, w2, b2, w3p, b3p)


def _edge_body(xl_hbm, xr_hbm, att_hbm, src_hbm, dst_hbm, eoff_hbm,
               out_hbm, lg_hbm, slab, rows, srcv, dstv, lgv, alv, attv,
               maxt, dent, eoffv, sem):
    cid = lax.axis_index("c")
    sid = lax.axis_index("s")
    wid = sid * 2 + cid
    lane = lax.iota(jnp.int32, 16)
    m0 = lane == 0

    pltpu.sync_copy(att_hbm, attv)
    pltpu.sync_copy(eoff_hbm, eoffv)

    def _sub(r, _carry):
        k = wid * 2 + r
        lo = k * NSEG
        nseg = jnp.minimum(NSEG, N - lo)
        hi = lo + nseg
        ev = eoffv[pl.ds(k, 16)]
        e_s = pl.multiple_of(ev[0], CH)
        e_e = ev[1]
        nch = (e_e - e_s) // CH

        pltpu.sync_copy(xr_hbm.at[pl.ds(lo, NSEG)], slab)

        def _init_tab(t, _):
            maxt[pl.ds(t * 16, 16)] = jnp.full((16,), -3e38, jnp.float32)
            dent[pl.ds(t * 16, 16)] = jnp.zeros((16,), jnp.float32)
            return 0
        lax.fori_loop(0, 160 // 16, _init_tab, 0)

        def _p1(ci, _):
            base = pl.multiple_of(e_s + ci * CH, CH)
            pltpu.sync_copy(src_hbm.at[pl.ds(base, CH)], srcv)
            pltpu.sync_copy(dst_hbm.at[pl.ds(base, CH + 16)], dstv)
            pltpu.async_copy(xl_hbm.at[srcv], rows, sem).wait()

            for g in range(CH // 16):
                def _e16(i16, lacc):
                    i = g * 16 + i16
                    d = dstv[pl.ds(i, 16)][0]
                    seg = jnp.clip(d - lo, 0, nseg - 1)
                    def _hc8(h8, acc):
                        for u in range(8):
                            o = h8 * 128 + u * 16
                            v = rows[i, pl.ds(o, 16)] + slab[seg, pl.ds(o, 16)]
                            v = jnp.maximum(v, 0.2 * v)
                            acc = acc + v * attv[pl.ds(o, 16)]
                        return acc
                    acc = lax.fori_loop(0, HC // 8, _hc8, jnp.zeros((16,), jnp.float32))
                    return jnp.where(lane == i16, jnp.sum(acc), lacc)
                lacc = lax.fori_loop(0, 16, _e16, jnp.zeros((16,), jnp.float32))
                lgv[pl.ds(g * 16, 16)] = lacc

            def _mx(i, _):
                d = dstv[pl.ds(i, 16)][0]
                seg = jnp.clip(d - lo, 0, nseg - 1)
                segv = jnp.full((16,), seg, jnp.int32)
                lg = lgv[pl.ds(i, 16)][0]
                lgs = jnp.where(d < hi, lg, -3e38)
                cur = plsc.load_gather(maxt, [segv])
                new = jnp.maximum(cur, jnp.full((16,), lgs, jnp.float32))
                plsc.store_scatter(maxt, [segv], new, mask=m0)
                return 0
            lax.fori_loop(0, CH, _mx, 0)
            pltpu.sync_copy(lgv.at[pl.ds(0, CH)], lg_hbm.at[pl.ds(base, CH)])
            return 0
        lax.fori_loop(0, nch, _p1, 0)

        def _p15(ci, _):
            base = pl.multiple_of(e_s + ci * CH, CH)
            pltpu.sync_copy(dst_hbm.at[pl.ds(base, CH + 16)], dstv)
            pltpu.sync_copy(lg_hbm.at[pl.ds(base, CH + 16)], lgv)

            def _dn(i, _):
                d = dstv[pl.ds(i, 16)][0]
                seg = jnp.clip(d - lo, 0, nseg - 1)
                segv = jnp.full((16,), seg, jnp.int32)
                lg = lgv[pl.ds(i, 16)][0]
                mx = plsc.load_gather(maxt, [segv])
                cur = plsc.load_gather(dent, [segv])
                p = jnp.where(d < hi, jnp.exp(jnp.full((16,), lg, jnp.float32) - mx), 0.0)
                plsc.store_scatter(dent, [segv], cur + p, mask=m0)
                return 0
            lax.fori_loop(0, CH, _dn, 0)
            return 0
        lax.fori_loop(0, nch, _p15, 0)

        def _zrow(rr, _):
            row = rr // 4
            cb = (rr % 4) * 128
            for u in range(8):
                slab[row, pl.ds(cb + u * 16, 16)] = jnp.zeros((16,), jnp.float32)
            return 0
        lax.fori_loop(0, NSEG * 4, _zrow, 0)

        def _p2(ci, _):
            base = pl.multiple_of(e_s + ci * CH, CH)
            pltpu.sync_copy(src_hbm.at[pl.ds(base, CH)], srcv)
            pltpu.sync_copy(dst_hbm.at[pl.ds(base, CH + 16)], dstv)
            pltpu.sync_copy(lg_hbm.at[pl.ds(base, CH + 16)], lgv)
            pltpu.async_copy(xl_hbm.at[srcv], rows, sem).wait()

            for g in range(CH // 16):
                d16 = dstv[pl.ds(g * 16, 16)]
                m = d16 < hi
                seg16 = jnp.clip(d16 - lo, 0, nseg - 1)
                mv = plsc.load_gather(maxt, [seg16])
                dv = plsc.load_gather(dent, [seg16])
                l16 = lgv[pl.ds(g * 16, 16)]
                t = jnp.where(m, l16 - mv, -30.0)
                a = jnp.where(m, jnp.exp(t) / dv, 0.0)
                alv[pl.ds(g * 16, 16)] = a

            def _acc(i, _):
                a = alv[pl.ds(i, 16)][0]
                d = dstv[pl.ds(i, 16)][0]
                seg = jnp.clip(d - lo, 0, nseg - 1)
                def _hc8a(h8, _):
                    for u in range(8):
                        o = h8 * 128 + u * 16
                        plsc.addupdate(slab.at[seg, pl.ds(o, 16)],
                                       a * rows[i, pl.ds(o, 16)])
                    return 0
                lax.fori_loop(0, HC // 8, _hc8a, 0)
                return 0
            lax.fori_loop(0, CH, _acc, 0)
            return 0
        lax.fori_loop(0, nch, _p2, 0)

        pltpu.sync_copy(slab, out_hbm.at[pl.ds(lo, NSEG)])
        return 0
    lax.fori_loop(0, 2, _sub, 0)


def _edge_stage_sc(xl, xr, att, srcp, dstp, eoff, lg_hbm_shape):
    mesh = plsc.VectorSubcoreMesh(core_axis_name="c", subcore_axis_name="s")
    f = pl.kernel(
        _edge_body,
        out_type=(jax.ShapeDtypeStruct((NP, H), jnp.float32),
                  jax.ShapeDtypeStruct((EPAD,), jnp.float32)),
        mesh=mesh,
        compiler_params=pltpu.CompilerParams(needs_layout_passes=False),
        scratch_types=[
            pltpu.VMEM((NSEG, H), jnp.float32),
            pltpu.VMEM((CH, H), jnp.float32),
            pltpu.VMEM((CH,), jnp.int32),
            pltpu.VMEM((CH + 16,), jnp.int32),
            pltpu.VMEM((CH + 16,), jnp.float32),
            pltpu.VMEM((CH + 16,), jnp.float32),
            pltpu.VMEM((H,), jnp.float32),
            pltpu.VMEM((160,), jnp.float32),
            pltpu.VMEM((160,), jnp.float32),
            pltpu.VMEM((96,), jnp.int32),
            pltpu.SemaphoreType.DMA,
        ],
    )
    out, _ = f(xl, xr, att, srcp, dstp, eoff)
    return out


def _prep_edges(edge_index):
    loops = jnp.arange(N, dtype=edge_index.dtype)
    src = jnp.concatenate([edge_index[0], loops])
    dst = jnp.concatenate([edge_index[1], loops])
    order = jnp.argsort(dst)
    src_s = src[order].astype(jnp.int32)
    dst_s = dst[order].astype(jnp.int32)
    bounds = (jnp.arange(NSUB + 1, dtype=jnp.int32) * NSEG).astype(dst_s.dtype)
    seg_start = jnp.searchsorted(dst_s, bounds[:-1], side="left").astype(jnp.int32)
    seg_end = jnp.searchsorted(dst_s, bounds[1:], side="left").astype(jnp.int32)
    cnt = seg_end - seg_start
    pcnt = ((cnt + CH - 1) // CH) * CH
    off = jnp.concatenate([jnp.zeros((1,), jnp.int32), jnp.cumsum(pcnt)]).astype(jnp.int32)
    bucket = jnp.clip(dst_s // NSEG, 0, NSUB - 1)
    pos = off[bucket] + (jnp.arange(E + N, dtype=jnp.int32) - seg_start[bucket])
    srcp = jnp.zeros((EPAD,), jnp.int32).at[pos].set(src_s)
    dstp = jnp.full((EPAD,), N, jnp.int32).at[pos].set(dst_s)
    eoff = jnp.concatenate([off, jnp.full((96 - NSUB - 1,), off[NSUB], jnp.int32)])
    return srcp, dstp, eoff


def kernel(x, edge_index, Wl1, Wr1, att1, b1, Wl2, Wr2, att2, b2,
           Wl3, Wr3, att3, b3, lW1, lb1, lW2, lb2, lW3, lb3):
    srcp, dstp, eoff = _prep_edges(edge_index)

    xp = jnp.pad(x, ((0, NP - N), (0, 0)))
    w1 = jnp.concatenate([Wl1, Wr1], axis=1)
    w2 = jnp.concatenate([Wl2, Wr2], axis=1)
    w3 = jnp.concatenate([Wl3, Wr3], axis=1)
    w3p = jnp.pad(lW3, ((0, 0), (0, 127)))
    b3p = jnp.pad(lb3, (0, 127))

    xl, xr = _proj(xp, w1)
    g1 = _edge_stage_sc(xl, xr, att1, srcp, dstp, eoff, None)
    xl, xr = _act_proj(g1, b1[None, :], w2)
    g2 = _edge_stage_sc(xl, xr, att2, srcp, dstp, eoff, None)
    xl, xr = _act_proj(g2, b2[None, :], w3)
    g3 = _edge_stage_sc(xl, xr, att3, srcp, dstp, eoff, None)
    y = _head(g3, b3[None, :], lW1, lb1[None, :], lW2, lb2[None, :],
              w3p, b3p[None, :])
    return y[:N, :1]
